# baseline (speedup 1.0000x reference)
import jax
import jax.numpy as jnp
from jax.experimental import pallas as pl
from jax.experimental.pallas import tpu as pltpu

_NCLS = 16
_W = 512
_H = 512
_HB = 128
_NH = _H // _HB
_R = 64
_NS = _HB // _R

_M2 = 0x33333333
_M4 = 0x0F0F0F0F
_M8 = 0x00FF00FF
_ARR = {0: 0, 2: 1, 1: 2, 3: 3}


def _make_kernel(nsteps):
    def _jaccard_kernel(pred_ref, tgt_ref, out_ref, wacc_ref, hist_ref):
        b = pl.program_id(0)
        accs = [[None] * 4 for _ in range(3)]

        for s in range(_NS):
            r0 = s * _R
            best = pred_ref[0, 0, r0:r0 + _R, :]
            oh = jnp.full((_R, _W), 1, jnp.int32)
            for c in range(1, _NCLS):
                xc = pred_ref[0, c, r0:r0 + _R, :]
                take = xc > best
                best = jnp.where(take, xc, best)
                oh = jnp.where(take, jnp.int32(1 << (2 * c)), oh)

            t = tgt_ref[0, r0:r0 + _R, :]
            oh_t_f = pltpu.bitcast(((t << 1) + 127) << 23, jnp.float32)
            oh_t = jnp.round(oh_t_f).astype(jnp.int32)
            oh_i = jnp.where(oh == oh_t, oh, jnp.int32(0))

            for chain, z in enumerate((oh, oh_i, oh_t)):
                a = z[0:8] + z[8:16] + z[16:24]
                bb = z[24:32] + z[32:40] + z[40:48]
                d = z[48:56] + z[56:64]
                e4 = (a & _M2) + (bb & _M2) + (d & _M2)
                o4 = ((a >> 2) & _M2) + ((bb >> 2) & _M2) + ((d >> 2) & _M2)
                u = (e4 & _M4,
                     (e4 >> 4) & _M4,
                     o4 & _M4,
                     (o4 >> 4) & _M4)
                for k in range(4):
                    accs[chain][k] = u[k] if s == 0 else accs[chain][k] + u[k]

        for chain in range(3):
            for k in range(4):
                a8 = accs[chain][k]
                lo = a8 & _M8
                hi = (a8 >> 8) & _M8
                wacc_ref[chain, 2 * k] = jnp.where(
                    b == 0, lo, wacc_ref[chain, 2 * k] + lo)
                wacc_ref[chain, 2 * k + 1] = jnp.where(
                    b == 0, hi, wacc_ref[chain, 2 * k + 1] + hi)

        @pl.when(b == nsteps - 1)
        def _finish():
            for chain in range(3):
                for c in range(_NCLS):
                    k = _ARR[c & 3]
                    m = c >> 2
                    arr = wacc_ref[chain, 2 * k + (m & 1)]
                    hist_ref[chain, c] = jnp.sum((arr >> (16 * (m >> 1)))
                                                 & 0xFFFF)
            tot = jnp.float32(0.0)
            for c in range(_NCLS):
                cp = hist_ref[0, c].astype(jnp.float32)
                ci = hist_ref[1, c].astype(jnp.float32)
                ct = hist_ref[2, c].astype(jnp.float32)
                union = cp + ct - ci
                tot = tot + jnp.where(
                    union == 0, jnp.float32(1.0),
                    ci / jnp.where(union == 0, jnp.float32(1.0), union))
            out_ref[0, 0] = tot * jnp.float32(1.0 / _NCLS)

    return _jaccard_kernel


def kernel(pred, target, interpret=False):
    nb = pred.shape[0]
    nsteps = nb * _NH
    out = pl.pallas_call(
        _make_kernel(nsteps),
        grid=(nsteps,),
        in_specs=[
            pl.BlockSpec((1, _NCLS, _HB, _W),
                         lambda i: (i // _NH, 0, i % _NH, 0)),
            pl.BlockSpec((1, _HB, _W), lambda i: (i // _NH, i % _NH, 0)),
        ],
        out_specs=pl.BlockSpec((1, 1), lambda i: (0, 0),
                               memory_space=pltpu.SMEM),
        out_shape=jax.ShapeDtypeStruct((1, 1), jnp.float32),
        scratch_shapes=[
            pltpu.VMEM((3, 8, 8, _W), jnp.int32),
            pltpu.SMEM((3, _NCLS), jnp.int32),
        ],
        compiler_params=pltpu.CompilerParams(
            dimension_semantics=("arbitrary",),
            vmem_limit_bytes=56 * 1024 * 1024),
        interpret=interpret,
    )(pred, target)
    return jnp.reshape(out, ())


# speedup vs baseline: 1.1298x; 1.1298x over previous
import jax
import jax.numpy as jnp
from jax.experimental import pallas as pl
from jax.experimental.pallas import tpu as pltpu

_NCLS = 16
_W = 512
_H = 512
_HB = 512
_NH = _H // _HB
_R = 64
_NS = _HB // _R

_M2 = 0x33333333
_M4 = 0x0F0F0F0F
_M8 = 0x00FF00FF
_ARR = {0: 0, 2: 1, 1: 2, 3: 3}


def _make_kernel(nsteps):
    def _jaccard_kernel(pred_ref, tgt_ref, out_ref, wacc_ref, hist_ref):
        b = pl.program_id(0)
        accs = [[None] * 4 for _ in range(3)]

        for s in range(_NS):
            r0 = s * _R
            best = pred_ref[0, 0, r0:r0 + _R, :]
            oh = jnp.full((_R, _W), 1, jnp.int32)
            for c in range(1, _NCLS):
                xc = pred_ref[0, c, r0:r0 + _R, :]
                take = xc > best
                best = jnp.where(take, xc, best)
                oh = jnp.where(take, jnp.int32(1 << (2 * c)), oh)

            t = tgt_ref[0, r0:r0 + _R, :]
            oh_t_f = pltpu.bitcast(((t << 1) + 127) << 23, jnp.float32)
            oh_t = jnp.round(oh_t_f).astype(jnp.int32)
            oh_i = jnp.where(oh == oh_t, oh, jnp.int32(0))

            for chain, z in enumerate((oh, oh_i, oh_t)):
                a = z[0:8] + z[8:16] + z[16:24]
                bb = z[24:32] + z[32:40] + z[40:48]
                d = z[48:56] + z[56:64]
                e4 = (a & _M2) + (bb & _M2) + (d & _M2)
                o4 = ((a >> 2) & _M2) + ((bb >> 2) & _M2) + ((d >> 2) & _M2)
                u = (e4 & _M4,
                     (e4 >> 4) & _M4,
                     o4 & _M4,
                     (o4 >> 4) & _M4)
                for k in range(4):
                    accs[chain][k] = u[k] if s == 0 else accs[chain][k] + u[k]

        for chain in range(3):
            for k in range(4):
                a8 = accs[chain][k]
                lo = a8 & _M8
                hi = (a8 >> 8) & _M8
                wacc_ref[chain, 2 * k] = jnp.where(
                    b == 0, lo, wacc_ref[chain, 2 * k] + lo)
                wacc_ref[chain, 2 * k + 1] = jnp.where(
                    b == 0, hi, wacc_ref[chain, 2 * k + 1] + hi)

        @pl.when(b == nsteps - 1)
        def _finish():
            for chain in range(3):
                for c in range(_NCLS):
                    k = _ARR[c & 3]
                    m = c >> 2
                    arr = wacc_ref[chain, 2 * k + (m & 1)]
                    hist_ref[chain, c] = jnp.sum((arr >> (16 * (m >> 1)))
                                                 & 0xFFFF)
            tot = jnp.float32(0.0)
            for c in range(_NCLS):
                cp = hist_ref[0, c].astype(jnp.float32)
                ci = hist_ref[1, c].astype(jnp.float32)
                ct = hist_ref[2, c].astype(jnp.float32)
                union = cp + ct - ci
                tot = tot + jnp.where(
                    union == 0, jnp.float32(1.0),
                    ci / jnp.where(union == 0, jnp.float32(1.0), union))
            out_ref[0, 0] = tot * jnp.float32(1.0 / _NCLS)

    return _jaccard_kernel


def kernel(pred, target, interpret=False):
    nb = pred.shape[0]
    nsteps = nb * _NH
    out = pl.pallas_call(
        _make_kernel(nsteps),
        grid=(nsteps,),
        in_specs=[
            pl.BlockSpec((1, _NCLS, _HB, _W),
                         lambda i: (i // _NH, 0, i % _NH, 0)),
            pl.BlockSpec((1, _HB, _W), lambda i: (i // _NH, i % _NH, 0)),
        ],
        out_specs=pl.BlockSpec((1, 1), lambda i: (0, 0),
                               memory_space=pltpu.SMEM),
        out_shape=jax.ShapeDtypeStruct((1, 1), jnp.float32),
        scratch_shapes=[
            pltpu.VMEM((3, 8, 8, _W), jnp.int32),
            pltpu.SMEM((3, _NCLS), jnp.int32),
        ],
        compiler_params=pltpu.CompilerParams(
            dimension_semantics=("arbitrary",),
            vmem_limit_bytes=56 * 1024 * 1024),
        interpret=interpret,
    )(pred, target)
    return jnp.reshape(out, ())


# speedup vs baseline: 1.1564x; 1.0235x over previous
import jax
import jax.numpy as jnp
from jax.experimental import pallas as pl
from jax.experimental.pallas import tpu as pltpu

_NCLS = 16
_W = 512
_H = 512
_HB = 256
_NH = _H // _HB
_R = 64
_NS = _HB // _R

_M2 = 0x33333333
_M4 = 0x0F0F0F0F
_M8 = 0x00FF00FF
_ARR = {0: 0, 2: 1, 1: 2, 3: 3}


def _make_kernel(nsteps):
    def _jaccard_kernel(pred_ref, tgt_ref, out_ref, wacc_ref, hist_ref):
        b = pl.program_id(0)
        accs = [[None] * 4 for _ in range(3)]

        for s in range(_NS):
            r0 = s * _R
            best = pred_ref[0, 0, r0:r0 + _R, :]
            oh = jnp.full((_R, _W), 1, jnp.int32)
            for c in range(1, _NCLS):
                xc = pred_ref[0, c, r0:r0 + _R, :]
                take = xc > best
                best = jnp.where(take, xc, best)
                oh = jnp.where(take, jnp.int32(1 << (2 * c)), oh)

            t = tgt_ref[0, r0:r0 + _R, :]
            oh_t_f = pltpu.bitcast(((t << 1) + 127) << 23, jnp.float32)
            oh_t = jnp.round(oh_t_f).astype(jnp.int32)
            oh_i = jnp.where(oh == oh_t, oh, jnp.int32(0))

            for chain, z in enumerate((oh, oh_i, oh_t)):
                a = z[0:8] + z[8:16] + z[16:24]
                bb = z[24:32] + z[32:40] + z[40:48]
                d = z[48:56] + z[56:64]
                e4 = (a & _M2) + (bb & _M2) + (d & _M2)
                o4 = ((a >> 2) & _M2) + ((bb >> 2) & _M2) + ((d >> 2) & _M2)
                u = (e4 & _M4,
                     (e4 >> 4) & _M4,
                     o4 & _M4,
                     (o4 >> 4) & _M4)
                for k in range(4):
                    accs[chain][k] = u[k] if s == 0 else accs[chain][k] + u[k]

        for chain in range(3):
            for k in range(4):
                a8 = accs[chain][k]
                lo = a8 & _M8
                hi = (a8 >> 8) & _M8
                wacc_ref[chain, 2 * k] = jnp.where(
                    b == 0, lo, wacc_ref[chain, 2 * k] + lo)
                wacc_ref[chain, 2 * k + 1] = jnp.where(
                    b == 0, hi, wacc_ref[chain, 2 * k + 1] + hi)

        @pl.when(b == nsteps - 1)
        def _finish():
            for chain in range(3):
                for c in range(_NCLS):
                    k = _ARR[c & 3]
                    m = c >> 2
                    arr = wacc_ref[chain, 2 * k + (m & 1)]
                    hist_ref[chain, c] = jnp.sum((arr >> (16 * (m >> 1)))
                                                 & 0xFFFF)
            tot = jnp.float32(0.0)
            for c in range(_NCLS):
                cp = hist_ref[0, c].astype(jnp.float32)
                ci = hist_ref[1, c].astype(jnp.float32)
                ct = hist_ref[2, c].astype(jnp.float32)
                union = cp + ct - ci
                tot = tot + jnp.where(
                    union == 0, jnp.float32(1.0),
                    ci / jnp.where(union == 0, jnp.float32(1.0), union))
            out_ref[0, 0] = tot * jnp.float32(1.0 / _NCLS)

    return _jaccard_kernel


def kernel(pred, target, interpret=False):
    nb = pred.shape[0]
    nsteps = nb * _NH
    out = pl.pallas_call(
        _make_kernel(nsteps),
        grid=(nsteps,),
        in_specs=[
            pl.BlockSpec((1, _NCLS, _HB, _W),
                         lambda i: (i // _NH, 0, i % _NH, 0)),
            pl.BlockSpec((1, _HB, _W), lambda i: (i // _NH, i % _NH, 0)),
        ],
        out_specs=pl.BlockSpec((1, 1), lambda i: (0, 0),
                               memory_space=pltpu.SMEM),
        out_shape=jax.ShapeDtypeStruct((1, 1), jnp.float32),
        scratch_shapes=[
            pltpu.VMEM((3, 8, 8, _W), jnp.int32),
            pltpu.SMEM((3, _NCLS), jnp.int32),
        ],
        compiler_params=pltpu.CompilerParams(
            dimension_semantics=("arbitrary",),
            vmem_limit_bytes=56 * 1024 * 1024),
        interpret=interpret,
    )(pred, target)
    return jnp.reshape(out, ())


# speedup vs baseline: 1.1570x; 1.0005x over previous
import jax
import jax.numpy as jnp
from jax.experimental import pallas as pl
from jax.experimental.pallas import tpu as pltpu

_NCLS = 16
_W = 512
_H = 512
_HB = 256
_NH = _H // _HB
_R = 64
_NS = _HB // _R

_M2 = 0x33333333
_M4 = 0x0F0F0F0F
_M8 = 0x00FF00FF
_ARR = {0: 0, 2: 1, 1: 2, 3: 3}


def _make_kernel(nsteps):
    def _jaccard_kernel(pred_ref, tgt_ref, out_ref, wacc_ref, hist_ref):
        b = pl.program_id(0)
        accs = [[None] * 4 for _ in range(3)]

        for s in range(_NS):
            r0 = s * _R
            best = pred_ref[0, 0, r0:r0 + _R, :]
            oh = jnp.full((_R, _W), 1, jnp.int32)
            for c in range(1, _NCLS):
                xc = pred_ref[0, c, r0:r0 + _R, :]
                take = xc > best
                best = jnp.where(take, xc, best)
                oh = jnp.where(take, jnp.int32(1 << (2 * c)), oh)

            t = tgt_ref[0, r0:r0 + _R, :]
            oh_t_f = pltpu.bitcast(((t << 1) + 127) << 23, jnp.float32)
            oh_t = jnp.round(oh_t_f).astype(jnp.int32)
            oh_i = jnp.where(oh == oh_t, oh, jnp.int32(0))

            for chain, z in enumerate((oh, oh_i, oh_t)):
                a = z[0:8] + z[8:16] + z[16:24]
                bb = z[24:32] + z[32:40] + z[40:48]
                d = z[48:56] + z[56:64]
                e4 = (a & _M2) + (bb & _M2) + (d & _M2)
                o4 = ((a >> 2) & _M2) + ((bb >> 2) & _M2) + ((d >> 2) & _M2)
                u = (e4 & _M4,
                     (e4 >> 4) & _M4,
                     o4 & _M4,
                     (o4 >> 4) & _M4)
                for k in range(4):
                    accs[chain][k] = u[k] if s == 0 else accs[chain][k] + u[k]

        for chain in range(3):
            for k in range(4):
                a8 = accs[chain][k]
                lo = a8 & _M8
                hi = (a8 >> 8) & _M8
                wacc_ref[chain, 2 * k] = jnp.where(
                    b == 0, lo, wacc_ref[chain, 2 * k] + lo)
                wacc_ref[chain, 2 * k + 1] = jnp.where(
                    b == 0, hi, wacc_ref[chain, 2 * k + 1] + hi)

        @pl.when(b == nsteps - 1)
        def _finish():
            for chain in range(3):
                folded = {}
                for q in range(8):
                    a = wacc_ref[chain, q]
                    folded[q] = (a[:, 0:128] + a[:, 128:256]
                                 + a[:, 256:384] + a[:, 384:512])
                for c in range(_NCLS):
                    k = _ARR[c & 3]
                    m = c >> 2
                    arr = folded[2 * k + (m & 1)]
                    hist_ref[chain, c] = jnp.sum((arr >> (16 * (m >> 1)))
                                                 & 0xFFFF)
            tot = jnp.float32(0.0)
            for c in range(_NCLS):
                cp = hist_ref[0, c].astype(jnp.float32)
                ci = hist_ref[1, c].astype(jnp.float32)
                ct = hist_ref[2, c].astype(jnp.float32)
                union = cp + ct - ci
                tot = tot + jnp.where(
                    union == 0, jnp.float32(1.0),
                    ci / jnp.where(union == 0, jnp.float32(1.0), union))
            out_ref[0, 0] = tot * jnp.float32(1.0 / _NCLS)

    return _jaccard_kernel


def kernel(pred, target, interpret=False):
    nb = pred.shape[0]
    nsteps = nb * _NH
    out = pl.pallas_call(
        _make_kernel(nsteps),
        grid=(nsteps,),
        in_specs=[
            pl.BlockSpec((1, _NCLS, _HB, _W),
                         lambda i: (i // _NH, 0, i % _NH, 0)),
            pl.BlockSpec((1, _HB, _W), lambda i: (i // _NH, i % _NH, 0)),
        ],
        out_specs=pl.BlockSpec((1, 1), lambda i: (0, 0),
                               memory_space=pltpu.SMEM),
        out_shape=jax.ShapeDtypeStruct((1, 1), jnp.float32),
        scratch_shapes=[
            pltpu.VMEM((3, 8, 8, _W), jnp.int32),
            pltpu.SMEM((3, _NCLS), jnp.int32),
        ],
        compiler_params=pltpu.CompilerParams(
            dimension_semantics=("arbitrary",),
            vmem_limit_bytes=56 * 1024 * 1024),
        interpret=interpret,
    )(pred, target)
    return jnp.reshape(out, ())
